# baseline (speedup 1.0000x reference)
import jax
import jax.numpy as jnp
from jax import lax
from jax.experimental import pallas as pl
from jax.experimental.pallas import tpu as pltpu
from jax.experimental.pallas import tpu_sc as plsc

SEQ = 200
DIM = 64
ROWPAD = 128
VOCAB = 1000000
BATCH = 4096
NC, NS, LANES = 2, 16, 16
NW = NC * NS
BPW = BATCH // NW
GA = 120
GB = SEQ - GA
NCH = 2 * BPW
NBUF = 4
K1_CB = 8192


def _table_widen(tT):
    def body(x_ref, o_ref):
        eye = jnp.eye(DIM, dtype=jnp.float32)
        eye2 = jnp.concatenate([eye, eye], axis=1)
        o_ref[...] = lax.dot_general(
            x_ref[...], eye2, (((0,), (0,)), ((), ())),
            precision=lax.Precision.HIGHEST)

    grid = (VOCAB + K1_CB - 1) // K1_CB
    return pl.pallas_call(
        body,
        grid=(grid,),
        in_specs=[pl.BlockSpec((DIM, K1_CB), lambda i: (0, i))],
        out_specs=pl.BlockSpec((K1_CB, ROWPAD), lambda i: (i, 0)),
        out_shape=jax.ShapeDtypeStruct((VOCAB, ROWPAD), jnp.float32),
    )(tT)


def _sc_body(table_hbm, idxa_hbm, idxb_hbm, pos_hbm, out_hbm,
             idxa_v, idxb_v, pos_v, rows_v, gsems, ssems):
    wid = lax.axis_index("s") * NC + lax.axis_index("c")
    bbase = wid * BPW

    pltpu.sync_copy(idxa_hbm.at[wid], idxa_v)
    pltpu.sync_copy(idxb_hbm.at[wid], idxb_v)
    pltpu.sync_copy(pos_hbm, pos_v)

    def gather_pair(c, half, b):
        s = c // 2
        n = GA if half == 0 else GB
        idx_src = idxa_v.at[s] if half == 0 else idxb_v.at[s]
        return (table_hbm.at[idx_src], rows_v.at[b, pl.ds(0, n)],
                gsems.at[b])

    def issue_gather(c, half, b):
        src, dst, sem = gather_pair(c, half, b)
        pltpu.async_copy(src, dst, sem)

    def wait_gather(c, half, b):
        src, dst, sem = gather_pair(c, half, b)
        pltpu.make_async_copy(src, dst, sem).wait()

    def store_pair(c, half, b):
        n = GA if half == 0 else GB
        row0 = (bbase + c // 2) * SEQ + half * GA
        return (rows_v.at[b, pl.ds(0, n)],
                out_hbm.at[pl.ds(row0, n)], ssems.at[b])

    def issue_store(c, half, b):
        src, dst, sem = store_pair(c, half, b)
        pltpu.async_copy(src, dst, sem)

    def wait_store(c, half, b):
        src, dst, sem = store_pair(c, half, b)
        pltpu.make_async_copy(src, dst, sem).wait()

    def process(c, half, b):
        wait_gather(c, half, b)
        n = GA if half == 0 else GB
        off = half * GA

        @pl.loop(0, n, unroll=2)
        def _add(r):
            for cc in range(DIM // LANES):
                sl = pl.ds(cc * LANES, LANES)
                plsc.addupdate(rows_v.at[b, r, sl], pos_v[off + r, sl])

        issue_store(c, half, b)

    for c in range(NBUF - 1):
        issue_gather(c, c % 2, c)

    @pl.loop(0, NCH, step=NBUF)
    def _outer(j):
        for b in range(NBUF):
            c = j + b
            half = b % 2
            process(c, half, b)

            nxt = c + NBUF - 1

            @pl.when(nxt < NCH)
            def _prefetch():
                @pl.when(c >= 1)
                def _drain():
                    wait_store(c - 1, (b + 1) % 2, (nxt) % NBUF)

                issue_gather(nxt, (b + 1) % 2, nxt % NBUF)

    for c in range(NCH - NBUF, NCH):
        wait_store(c, c % 2, c % NBUF)


def _sc_gather_add(table_wide, idxa, idxb, pos_table):
    mesh = plsc.VectorSubcoreMesh(core_axis_name="c", subcore_axis_name="s")
    run = pl.kernel(
        _sc_body,
        out_type=jax.ShapeDtypeStruct((BATCH * SEQ, ROWPAD), jnp.float32),
        mesh=mesh,
        scratch_types=[
            pltpu.VMEM((BPW, GA), jnp.int32),
            pltpu.VMEM((BPW, GB), jnp.int32),
            pltpu.VMEM((SEQ, DIM), jnp.float32),
            pltpu.VMEM((NBUF, GA, ROWPAD), jnp.float32),
            pltpu.SemaphoreType.DMA((NBUF,)),
            pltpu.SemaphoreType.DMA((NBUF,)),
        ],
        compiler_params=pltpu.CompilerParams(use_tc_tiling_on_sc=False),
    )
    return run(table_wide, idxa, idxb, pos_table)


@jax.jit
def _pos_embed(inputs, term_table, pos_table):
    idx = inputs.astype(jnp.int32).reshape(NW, BPW, SEQ)
    table_wide = _table_widen(term_table.T)
    wide = _sc_gather_add(table_wide, idx[:, :, :GA], idx[:, :, GA:],
                          pos_table)
    return wide[:, :DIM].reshape(BATCH, SEQ, DIM)


def kernel(inputs, term_table, pos_table):
    return _pos_embed(inputs, term_table, pos_table)


# speedup vs baseline: 1.0058x; 1.0058x over previous
import jax
import jax.numpy as jnp
from jax import lax
from jax.experimental import pallas as pl
from jax.experimental.pallas import tpu as pltpu
from jax.experimental.pallas import tpu_sc as plsc

SEQ = 200
DIM = 64
ROWPAD = 128
VOCAB = 1000000
BATCH = 4096
NC, NS, LANES = 2, 16, 16
NW = NC * NS
BPW = BATCH // NW
GA = 120
GB = SEQ - GA
NCH = 2 * BPW
NBUF = 4
K1_CB = 10240


def _table_widen(tT):
    def body(x_ref, o_ref):
        eye = jnp.eye(DIM, dtype=jnp.float32)
        eye2 = jnp.concatenate([eye, eye], axis=1)
        o_ref[...] = lax.dot_general(
            x_ref[...], eye2, (((0,), (0,)), ((), ())),
            precision=lax.Precision.HIGHEST)

    grid = (VOCAB + K1_CB - 1) // K1_CB
    return pl.pallas_call(
        body,
        grid=(grid,),
        in_specs=[pl.BlockSpec((DIM, K1_CB), lambda i: (0, i))],
        out_specs=pl.BlockSpec((K1_CB, ROWPAD), lambda i: (i, 0)),
        out_shape=jax.ShapeDtypeStruct((VOCAB, ROWPAD), jnp.float32),
    )(tT)


def _sc_body(table_hbm, idxa_hbm, idxb_hbm, pos_hbm, out_hbm,
             idxa_v, idxb_v, pos_v, rows_v, gsems, ssems):
    wid = lax.axis_index("s") * NC + lax.axis_index("c")
    bbase = wid * BPW

    pltpu.sync_copy(idxa_hbm.at[wid], idxa_v)
    pltpu.sync_copy(idxb_hbm.at[wid], idxb_v)
    pltpu.sync_copy(pos_hbm, pos_v)

    def gather_pair(c, half, b):
        s = c // 2
        n = GA if half == 0 else GB
        idx_src = idxa_v.at[s] if half == 0 else idxb_v.at[s]
        return (table_hbm.at[idx_src], rows_v.at[b, pl.ds(0, n)],
                gsems.at[b])

    def issue_gather(c, half, b):
        src, dst, sem = gather_pair(c, half, b)
        pltpu.async_copy(src, dst, sem)

    def wait_gather(c, half, b):
        src, dst, sem = gather_pair(c, half, b)
        pltpu.make_async_copy(src, dst, sem).wait()

    def store_pair(c, half, b):
        n = GA if half == 0 else GB
        row0 = (bbase + c // 2) * SEQ + half * GA
        return (rows_v.at[b, pl.ds(0, n)],
                out_hbm.at[pl.ds(row0, n)], ssems.at[b])

    def issue_store(c, half, b):
        src, dst, sem = store_pair(c, half, b)
        pltpu.async_copy(src, dst, sem)

    def wait_store(c, half, b):
        src, dst, sem = store_pair(c, half, b)
        pltpu.make_async_copy(src, dst, sem).wait()

    def process(c, half, b):
        wait_gather(c, half, b)
        n = GA if half == 0 else GB
        off = half * GA

        @pl.loop(0, n, unroll=2)
        def _add(r):
            for cc in range(DIM // LANES):
                sl = pl.ds(cc * LANES, LANES)
                plsc.addupdate(rows_v.at[b, r, sl], pos_v[off + r, sl])

        issue_store(c, half, b)

    for c in range(NBUF - 1):
        issue_gather(c, c % 2, c)

    @pl.loop(0, NCH, step=NBUF)
    def _outer(j):
        for b in range(NBUF):
            c = j + b
            half = b % 2
            process(c, half, b)

            nxt = c + NBUF - 1

            @pl.when(nxt < NCH)
            def _prefetch():
                @pl.when(c >= 1)
                def _drain():
                    wait_store(c - 1, (b + 1) % 2, (nxt) % NBUF)

                issue_gather(nxt, (b + 1) % 2, nxt % NBUF)

    for c in range(NCH - NBUF, NCH):
        wait_store(c, c % 2, c % NBUF)


def _sc_gather_add(table_wide, idxa, idxb, pos_table):
    mesh = plsc.VectorSubcoreMesh(core_axis_name="c", subcore_axis_name="s")
    run = pl.kernel(
        _sc_body,
        out_type=jax.ShapeDtypeStruct((BATCH * SEQ, ROWPAD), jnp.float32),
        mesh=mesh,
        scratch_types=[
            pltpu.VMEM((BPW, GA), jnp.int32),
            pltpu.VMEM((BPW, GB), jnp.int32),
            pltpu.VMEM((SEQ, DIM), jnp.float32),
            pltpu.VMEM((NBUF, GA, ROWPAD), jnp.float32),
            pltpu.SemaphoreType.DMA((NBUF,)),
            pltpu.SemaphoreType.DMA((NBUF,)),
        ],
        compiler_params=pltpu.CompilerParams(use_tc_tiling_on_sc=False),
    )
    return run(table_wide, idxa, idxb, pos_table)


@jax.jit
def _pos_embed(inputs, term_table, pos_table):
    idx = inputs.astype(jnp.int32).reshape(NW, BPW, SEQ)
    table_wide = _table_widen(term_table.T)
    wide = _sc_gather_add(table_wide, idx[:, :, :GA], idx[:, :, GA:],
                          pos_table)
    return wide[:, :DIM].reshape(BATCH, SEQ, DIM)


def kernel(inputs, term_table, pos_table):
    return _pos_embed(inputs, term_table, pos_table)
